# baseline (speedup 1.0000x reference)
import jax
import jax.numpy as jnp
from jax import lax
from jax.experimental import pallas as pl
from jax.experimental.pallas import tpu as pltpu
from jax.experimental.pallas import tpu_sc as plsc

N = 10000
E = 320000
F_IN = 128
HID = 16
CLS = 40
CLSP = 48

NC, NS, L = 2, 16, 16
NW = NC * NS
CHUNK = 128
NCHUNK = E // CHUNK
CPWB = NCHUNK // NW
NPAD = 10240
RPT = NPAD // NS


def _sc_mesh():
    return plsc.VectorSubcoreMesh(
        core_axis_name="c", subcore_axis_name="s", num_cores=NC, num_subcores=NS
    )


_SC_PARAMS = pltpu.CompilerParams(use_tc_tiling_on_sc=False)


def _make_propagate(F, nbuf):
    ng = (CPWB + 1 + 2 * nbuf - 1) // (2 * nbuf)

    def body(feat_hbm, edge_hbm, out_hbm, acc, src_v, dst_v, zbuf, rows, gsem, ssem):
        c = lax.axis_index("c")
        s = lax.axis_index("s")
        wid = s * NC + c
        cnt = jnp.where(wid < NCHUNK - NW * CPWB, CPWB + 1, CPWB)
        start = CPWB * wid + jnp.minimum(wid, NCHUNK - NW * CPWB)
        sl = pl.ds(s * RPT, RPT)

        def zfill(i, carry):
            for k in range(F // L):
                zbuf[i, pl.ds(k * L, L)] = jnp.zeros((L,), jnp.float32)
            return carry

        lax.fori_loop(0, 128, zfill, 0)
        for k in range(RPT // 128):
            pltpu.sync_copy(zbuf, acc.at[pl.ds(s * RPT + k * 128, 128)])
        pltpu.sync_copy(edge_hbm.at[0, pl.ds(start, CPWB)], src_v.at[pl.ds(0, CPWB)])
        pltpu.sync_copy(edge_hbm.at[1, pl.ds(start, CPWB)], dst_v.at[pl.ds(0, CPWB)])

        @pl.when(cnt > CPWB)
        def _():
            pltpu.sync_copy(edge_hbm.at[0, start + CPWB], src_v.at[CPWB])
            pltpu.sync_copy(edge_hbm.at[1, start + CPWB], dst_v.at[CPWB])

        plsc.subcore_barrier()

        def super_body(u, carry):
            base = u * 2 * nbuf
            ga = [
                pltpu.async_copy(
                    feat_hbm.at[src_v.at[base + i]], rows[0][i], gsem[0][i]
                )
                for i in range(nbuf)
            ]
            sa = []
            for i in range(nbuf):
                ga[i].wait()
                sa.append(
                    pltpu.async_copy(
                        rows[0][i], acc.at[dst_v.at[base + i]], ssem[0][i], add=True
                    )
                )
            gb = [
                pltpu.async_copy(
                    feat_hbm.at[src_v.at[base + nbuf + i]], rows[1][i], gsem[1][i]
                )
                for i in range(nbuf)
            ]
            for d in sa:
                d.wait()
            sb = []
            for i in range(nbuf):
                gb[i].wait()
                sb.append(
                    pltpu.async_copy(
                        rows[1][i], acc.at[dst_v.at[base + nbuf + i]], ssem[1][i],
                        add=True,
                    )
                )
            for d in sb:
                d.wait()
            return carry

        lax.fori_loop(0, ng - 1, super_body, 0)

        base = (ng - 1) * 2 * nbuf
        gds = [None] * (2 * nbuf)
        for i in range(2 * nbuf):
            def gissue(i=i):
                gds[i] = pltpu.async_copy(
                    feat_hbm.at[src_v.at[base + i]], rows[i // nbuf][i % nbuf],
                    gsem[i // nbuf][i % nbuf],
                )
            pl.when(base + i < cnt)(gissue)
        for i in range(2 * nbuf):
            def gwait(i=i):
                gds[i].wait()
                pltpu.sync_copy(
                    rows[i // nbuf][i % nbuf], acc.at[dst_v.at[base + i]], add=True
                )
            pl.when(base + i < cnt)(gwait)

        plsc.subcore_barrier()
        pltpu.sync_copy(acc.at[sl], out_hbm.at[c, sl])

    return pl.kernel(
        body,
        out_type=jax.ShapeDtypeStruct((NC, NPAD, F), jnp.float32),
        mesh=_sc_mesh(),
        compiler_params=_SC_PARAMS,
        scratch_types=[
            pltpu.VMEM_SHARED((NPAD, F), jnp.float32),
            pltpu.VMEM((ng * 2 * nbuf, CHUNK), jnp.int32),
            pltpu.VMEM((ng * 2 * nbuf, CHUNK), jnp.int32),
            pltpu.VMEM((128, F), jnp.float32),
            [[pltpu.VMEM((CHUNK, F), jnp.float32) for _ in range(nbuf)] for _ in range(2)],
            [[pltpu.SemaphoreType.DMA for _ in range(nbuf)] for _ in range(2)],
            [[pltpu.SemaphoreType.DMA for _ in range(nbuf)] for _ in range(2)],
        ],
    )


def _make_degree():
    def body(edge_hbm, out_hbm, acc, dst_v, ones_v, zbuf):
        c = lax.axis_index("c")
        s = lax.axis_index("s")
        wid = s * NC + c
        sl = pl.ds(s * RPT, RPT)

        def zfill(i, carry):
            zbuf[i, :] = jnp.zeros((L,), jnp.float32)
            return carry

        lax.fori_loop(0, 128, zfill, 0)
        for k in range(RPT // 128):
            pltpu.sync_copy(zbuf, acc.at[pl.ds(s * RPT + k * 128, 128)])

        def ones_body(i, carry):
            ones_v[i, :] = jnp.ones((L,), jnp.float32)
            return carry

        lax.fori_loop(0, CHUNK, ones_body, 0)
        cnt = jnp.where(wid < NCHUNK - NW * CPWB, CPWB + 1, CPWB)
        start = CPWB * wid + jnp.minimum(wid, NCHUNK - NW * CPWB)
        pltpu.sync_copy(edge_hbm.at[1, pl.ds(start, CPWB)], dst_v.at[pl.ds(0, CPWB)])

        @pl.when(cnt > CPWB)
        def _():
            pltpu.sync_copy(edge_hbm.at[1, start + CPWB], dst_v.at[CPWB])

        plsc.subcore_barrier()

        def jbody(j, carry):
            pltpu.sync_copy(ones_v, acc.at[dst_v.at[j]], add=True)
            return carry

        lax.fori_loop(0, CPWB, jbody, 0)

        @pl.when(cnt > CPWB)
        def _():
            pltpu.sync_copy(ones_v, acc.at[dst_v.at[CPWB]], add=True)

        plsc.subcore_barrier()
        pltpu.sync_copy(acc.at[sl], out_hbm.at[c, sl])

    return pl.kernel(
        body,
        out_type=jax.ShapeDtypeStruct((NC, NPAD, L), jnp.float32),
        mesh=_sc_mesh(),
        compiler_params=_SC_PARAMS,
        scratch_types=[
            pltpu.VMEM_SHARED((NPAD, L), jnp.float32),
            pltpu.VMEM((80, CHUNK), jnp.int32),
            pltpu.VMEM((CHUNK, L), jnp.float32),
            pltpu.VMEM((128, L), jnp.float32),
        ],
    )


_propagate16 = _make_propagate(HID, nbuf=4)
_propagate48 = _make_propagate(CLSP, nbuf=4)
_degree = _make_degree()

_RB = 128
_GRID = NPAD // (8 * _RB)
_XR = N * F_IN // 1024
_R16 = NPAD * HID // 128
_W48 = 8 * CLSP


def _stage1(xv, bdw1, degv):
    def body(xv_ref, w_ref, degv_ref, hs_ref, d16_ref):
        d16 = lax.rsqrt(degv_ref[0] + degv_ref[1] + 1.0)
        h = jnp.dot(xv_ref[...], w_ref[...], preferred_element_type=jnp.float32)
        hs_ref[...] = h * d16
        d16_ref[...] = d16

    return pl.pallas_call(
        body,
        grid=(_GRID,),
        in_specs=[
            pl.BlockSpec((_RB, 1024), lambda i: (i, 0)),
            pl.BlockSpec((1024, 128), lambda i: (0, 0)),
            pl.BlockSpec((NC, _RB, 128), lambda i: (0, i, 0)),
        ],
        out_specs=[
            pl.BlockSpec((_RB, 128), lambda i: (i, 0)),
            pl.BlockSpec((_RB, 128), lambda i: (i, 0)),
        ],
        out_shape=[
            jax.ShapeDtypeStruct((_R16, 128), jnp.float32),
            jax.ShapeDtypeStruct((_R16, 128), jnp.float32),
        ],
    )(xv, bdw1, degv)


def _stage2(p1v, hs_pk, d16, b1t, bdw2, m48):
    def body(p_ref, hs_ref, d16_ref, b1_ref, w_ref, m48_ref, hs2_ref):
        d16 = d16_ref[...]
        sacc = d16 * (p_ref[0] + p_ref[1] + hs_ref[...]) + b1_ref[...]
        h1 = jnp.maximum(sacc, 0.0)
        t = jnp.dot(h1, w_ref[...], preferred_element_type=jnp.float32)
        d48 = jnp.dot(d16, m48_ref[...], preferred_element_type=jnp.float32)
        hs2_ref[...] = t * d48

    return pl.pallas_call(
        body,
        grid=(_GRID,),
        in_specs=[
            pl.BlockSpec((NC, _RB, 128), lambda i: (0, i, 0)),
            pl.BlockSpec((_RB, 128), lambda i: (i, 0)),
            pl.BlockSpec((_RB, 128), lambda i: (i, 0)),
            pl.BlockSpec((1, 128), lambda i: (0, 0)),
            pl.BlockSpec((128, _W48), lambda i: (0, 0)),
            pl.BlockSpec((128, _W48), lambda i: (0, 0)),
        ],
        out_specs=pl.BlockSpec((_RB, _W48), lambda i: (i, 0)),
        out_shape=jax.ShapeDtypeStruct((_R16, _W48), jnp.float32),
    )(p1v, hs_pk, d16, b1t, bdw2, m48)


def _stage3(p2v, hs2_pk, d16, b2t, seg, s012, m48):
    def body(p_ref, hs2_ref, d16_ref, b2_ref, seg_ref, s_ref, m48_ref, *outs):
        x = p_ref[0] + p_ref[1]
        y = jnp.dot(s_ref[...], x, preferred_element_type=jnp.float32)
        p_pk = jnp.concatenate([y[0:128], y[128:256], y[256:384]], axis=1)
        d48 = jnp.dot(d16_ref[...], m48_ref[...], preferred_element_type=jnp.float32)
        tr = d48 * (p_pk + hs2_ref[...]) + b2_ref[...]
        m = jnp.max(tr, axis=1, keepdims=True)
        e = jnp.exp(tr - m)
        ssum = jnp.dot(e, seg_ref[...], preferred_element_type=jnp.float32)
        outs[0][...] = e / ssum
        outs[1][...] = tr

    o_spec = pl.BlockSpec((_RB, _W48), lambda i: (i, 0))
    return pl.pallas_call(
        body,
        grid=(_GRID,),
        in_specs=[
            pl.BlockSpec((NC, 3 * _RB, 128), lambda i: (0, i, 0)),
            pl.BlockSpec((_RB, _W48), lambda i: (i, 0)),
            pl.BlockSpec((_RB, 128), lambda i: (i, 0)),
            pl.BlockSpec((1, _W48), lambda i: (0, 0)),
            pl.BlockSpec((_W48, _W48), lambda i: (0, 0)),
            pl.BlockSpec((_W48, _W48), lambda i: (0, 0)),
            pl.BlockSpec((128, _W48), lambda i: (0, 0)),
        ],
        out_specs=[o_spec] * 2,
        out_shape=[jax.ShapeDtypeStruct((_R16, _W48), jnp.float32)] * 2,
    )(p2v, hs2_pk, d16, b2t, seg, s012, m48)


def kernel(x, edge_index, W1, b1, W2, b2):
    ev = edge_index.reshape(2, NCHUNK, CHUNK)
    xv = x.reshape(_XR, 1024)
    eye8 = jnp.eye(8, dtype=jnp.float32)
    bdw1 = jnp.kron(eye8, W1)
    w2p = jnp.zeros((HID, CLSP), jnp.float32).at[:, :CLS].set(W2)
    bdw2 = jnp.kron(eye8, w2p)
    pick = jnp.zeros((HID, CLSP), jnp.float32).at[0, :].set(1.0)
    m48 = jnp.kron(eye8, pick)
    b1t = jnp.tile(b1, 8).reshape(1, 128)
    b2p = jnp.full((CLSP,), -1e30, jnp.float32).at[:CLS].set(b2)
    b2t = jnp.tile(b2p, 8).reshape(1, _W48)
    seg = jnp.kron(eye8, jnp.ones((CLSP, CLSP), jnp.float32))
    ii = jnp.arange(_W48)
    s012 = (ii[None, :] == (3 * (ii % 128) + ii // 128)[:, None]).astype(jnp.float32)

    degp = _degree(ev)
    degv = degp.reshape(NC, _R16, 128)
    hs_pk, d16 = _stage1(xv, bdw1, degv)

    p1 = _propagate16(hs_pk.reshape(NPAD, HID), ev)
    p1v = p1.reshape(NC, _R16, 128)
    hs2_pk = _stage2(p1v, hs_pk, d16, b1t, bdw2, m48)

    p2 = _propagate48(hs2_pk.reshape(NPAD, CLSP), ev)
    p2v = p2.reshape(NC, 3 * _R16, 128)
    out_pk, logit_pk = _stage3(p2v, hs2_pk, d16, b2t, seg, s012, m48)
    out = out_pk.reshape(NPAD, CLSP)[:N, :CLS]
    logits = logit_pk.reshape(NPAD, CLSP)[:N, :CLS]
    return (out, logits)


# speedup vs baseline: 1.0316x; 1.0316x over previous
import jax
import jax.numpy as jnp
from jax import lax
from jax.experimental import pallas as pl
from jax.experimental.pallas import tpu as pltpu
from jax.experimental.pallas import tpu_sc as plsc

N = 10000
E = 320000
F_IN = 128
HID = 16
CLS = 40
CLSP = 48

NC, NS, L = 2, 16, 16
NW = NC * NS
CHUNK = 128
NCHUNK = E // CHUNK
CPWB = NCHUNK // NW
NPAD = 10240
RPT = NPAD // NS


def _sc_mesh():
    return plsc.VectorSubcoreMesh(
        core_axis_name="c", subcore_axis_name="s", num_cores=NC, num_subcores=NS
    )


_SC_PARAMS = pltpu.CompilerParams(use_tc_tiling_on_sc=False)


def _make_propagate(F, nbuf):
    ng = (CPWB + 1 + 2 * nbuf - 1) // (2 * nbuf)

    def body(feat_hbm, edge_hbm, out_hbm, acc, src_v, dst_v, zbuf, rows, gsem, ssem):
        c = lax.axis_index("c")
        s = lax.axis_index("s")
        wid = s * NC + c
        cnt = jnp.where(wid < NCHUNK - NW * CPWB, CPWB + 1, CPWB)
        start = CPWB * wid + jnp.minimum(wid, NCHUNK - NW * CPWB)
        sl = pl.ds(s * RPT, RPT)

        def zfill(i, carry):
            for k in range(F // L):
                zbuf[i, pl.ds(k * L, L)] = jnp.zeros((L,), jnp.float32)
            return carry

        lax.fori_loop(0, 128, zfill, 0)
        for k in range(RPT // 128):
            pltpu.sync_copy(zbuf, acc.at[pl.ds(s * RPT + k * 128, 128)])
        pltpu.sync_copy(edge_hbm.at[0, pl.ds(start, CPWB)], src_v.at[pl.ds(0, CPWB)])
        pltpu.sync_copy(edge_hbm.at[1, pl.ds(start, CPWB)], dst_v.at[pl.ds(0, CPWB)])

        @pl.when(cnt > CPWB)
        def _():
            pltpu.sync_copy(edge_hbm.at[0, start + CPWB], src_v.at[CPWB])
            pltpu.sync_copy(edge_hbm.at[1, start + CPWB], dst_v.at[CPWB])

        plsc.subcore_barrier()

        def super_body(u, carry):
            base = u * 2 * nbuf
            gds = [
                pltpu.async_copy(
                    feat_hbm.at[src_v.at[base + i]], rows[i // nbuf][i % nbuf],
                    gsem[i // nbuf][i % nbuf],
                )
                for i in range(2 * nbuf)
            ]
            sds = []
            for i in range(2 * nbuf):
                gds[i].wait()
                sds.append(
                    pltpu.async_copy(
                        rows[i // nbuf][i % nbuf], acc.at[dst_v.at[base + i]],
                        ssem[i // nbuf][i % nbuf], add=True,
                    )
                )
            for d in sds:
                d.wait()
            return carry

        lax.fori_loop(0, ng - 1, super_body, 0)

        base = (ng - 1) * 2 * nbuf
        gds = [None] * (2 * nbuf)
        for i in range(2 * nbuf):
            def gissue(i=i):
                gds[i] = pltpu.async_copy(
                    feat_hbm.at[src_v.at[base + i]], rows[i // nbuf][i % nbuf],
                    gsem[i // nbuf][i % nbuf],
                )
            pl.when(base + i < cnt)(gissue)
        for i in range(2 * nbuf):
            def gwait(i=i):
                gds[i].wait()
                pltpu.sync_copy(
                    rows[i // nbuf][i % nbuf], acc.at[dst_v.at[base + i]], add=True
                )
            pl.when(base + i < cnt)(gwait)

        plsc.subcore_barrier()
        pltpu.sync_copy(acc.at[sl], out_hbm.at[c, sl])

    return pl.kernel(
        body,
        out_type=jax.ShapeDtypeStruct((NC, NPAD, F), jnp.float32),
        mesh=_sc_mesh(),
        compiler_params=_SC_PARAMS,
        scratch_types=[
            pltpu.VMEM_SHARED((NPAD, F), jnp.float32),
            pltpu.VMEM((ng * 2 * nbuf, CHUNK), jnp.int32),
            pltpu.VMEM((ng * 2 * nbuf, CHUNK), jnp.int32),
            pltpu.VMEM((128, F), jnp.float32),
            [[pltpu.VMEM((CHUNK, F), jnp.float32) for _ in range(nbuf)] for _ in range(2)],
            [[pltpu.SemaphoreType.DMA for _ in range(nbuf)] for _ in range(2)],
            [[pltpu.SemaphoreType.DMA for _ in range(nbuf)] for _ in range(2)],
        ],
    )


def _make_degree():
    def body(edge_hbm, out_hbm, acc, dst_v, ones_v, zbuf):
        c = lax.axis_index("c")
        s = lax.axis_index("s")
        wid = s * NC + c
        sl = pl.ds(s * RPT, RPT)

        def zfill(i, carry):
            zbuf[i, :] = jnp.zeros((L,), jnp.float32)
            return carry

        lax.fori_loop(0, 128, zfill, 0)
        for k in range(RPT // 128):
            pltpu.sync_copy(zbuf, acc.at[pl.ds(s * RPT + k * 128, 128)])

        def ones_body(i, carry):
            ones_v[i, :] = jnp.ones((L,), jnp.float32)
            return carry

        lax.fori_loop(0, CHUNK, ones_body, 0)
        cnt = jnp.where(wid < NCHUNK - NW * CPWB, CPWB + 1, CPWB)
        start = CPWB * wid + jnp.minimum(wid, NCHUNK - NW * CPWB)
        pltpu.sync_copy(edge_hbm.at[1, pl.ds(start, CPWB)], dst_v.at[pl.ds(0, CPWB)])

        @pl.when(cnt > CPWB)
        def _():
            pltpu.sync_copy(edge_hbm.at[1, start + CPWB], dst_v.at[CPWB])

        plsc.subcore_barrier()

        def jbody(j, carry):
            pltpu.sync_copy(ones_v, acc.at[dst_v.at[j]], add=True)
            return carry

        lax.fori_loop(0, CPWB, jbody, 0)

        @pl.when(cnt > CPWB)
        def _():
            pltpu.sync_copy(ones_v, acc.at[dst_v.at[CPWB]], add=True)

        plsc.subcore_barrier()
        pltpu.sync_copy(acc.at[sl], out_hbm.at[c, sl])

    return pl.kernel(
        body,
        out_type=jax.ShapeDtypeStruct((NC, NPAD, L), jnp.float32),
        mesh=_sc_mesh(),
        compiler_params=_SC_PARAMS,
        scratch_types=[
            pltpu.VMEM_SHARED((NPAD, L), jnp.float32),
            pltpu.VMEM((80, CHUNK), jnp.int32),
            pltpu.VMEM((CHUNK, L), jnp.float32),
            pltpu.VMEM((128, L), jnp.float32),
        ],
    )


_propagate16 = _make_propagate(HID, nbuf=4)
_propagate48 = _make_propagate(CLSP, nbuf=4)
_degree = _make_degree()

_RB = 128
_GRID = NPAD // (8 * _RB)
_XR = N * F_IN // 1024
_R16 = NPAD * HID // 128
_W48 = 8 * CLSP


def _stage1(xv, bdw1, degv):
    def body(xv_ref, w_ref, degv_ref, hs_ref, d16_ref):
        d16 = lax.rsqrt(degv_ref[0] + degv_ref[1] + 1.0)
        h = jnp.dot(xv_ref[...], w_ref[...], preferred_element_type=jnp.float32)
        hs_ref[...] = h * d16
        d16_ref[...] = d16

    return pl.pallas_call(
        body,
        grid=(_GRID,),
        in_specs=[
            pl.BlockSpec((_RB, 1024), lambda i: (i, 0)),
            pl.BlockSpec((1024, 128), lambda i: (0, 0)),
            pl.BlockSpec((NC, _RB, 128), lambda i: (0, i, 0)),
        ],
        out_specs=[
            pl.BlockSpec((_RB, 128), lambda i: (i, 0)),
            pl.BlockSpec((_RB, 128), lambda i: (i, 0)),
        ],
        out_shape=[
            jax.ShapeDtypeStruct((_R16, 128), jnp.float32),
            jax.ShapeDtypeStruct((_R16, 128), jnp.float32),
        ],
    )(xv, bdw1, degv)


def _stage2(p1v, hs_pk, d16, b1t, bdw2, m48):
    def body(p_ref, hs_ref, d16_ref, b1_ref, w_ref, m48_ref, hs2_ref):
        d16 = d16_ref[...]
        sacc = d16 * (p_ref[0] + p_ref[1] + hs_ref[...]) + b1_ref[...]
        h1 = jnp.maximum(sacc, 0.0)
        t = jnp.dot(h1, w_ref[...], preferred_element_type=jnp.float32)
        d48 = jnp.dot(d16, m48_ref[...], preferred_element_type=jnp.float32)
        hs2_ref[...] = t * d48

    return pl.pallas_call(
        body,
        grid=(_GRID,),
        in_specs=[
            pl.BlockSpec((NC, _RB, 128), lambda i: (0, i, 0)),
            pl.BlockSpec((_RB, 128), lambda i: (i, 0)),
            pl.BlockSpec((_RB, 128), lambda i: (i, 0)),
            pl.BlockSpec((1, 128), lambda i: (0, 0)),
            pl.BlockSpec((128, _W48), lambda i: (0, 0)),
            pl.BlockSpec((128, _W48), lambda i: (0, 0)),
        ],
        out_specs=pl.BlockSpec((_RB, _W48), lambda i: (i, 0)),
        out_shape=jax.ShapeDtypeStruct((_R16, _W48), jnp.float32),
    )(p1v, hs_pk, d16, b1t, bdw2, m48)


def _stage3(p2v, hs2_pk, d16, b2t, seg, s012, m48):
    def body(p_ref, hs2_ref, d16_ref, b2_ref, seg_ref, s_ref, m48_ref, *outs):
        x = p_ref[0] + p_ref[1]
        y = jnp.dot(s_ref[...], x, preferred_element_type=jnp.float32)
        p_pk = jnp.concatenate([y[0:128], y[128:256], y[256:384]], axis=1)
        d48 = jnp.dot(d16_ref[...], m48_ref[...], preferred_element_type=jnp.float32)
        tr = d48 * (p_pk + hs2_ref[...]) + b2_ref[...]
        m = jnp.max(tr, axis=1, keepdims=True)
        e = jnp.exp(tr - m)
        ssum = jnp.dot(e, seg_ref[...], preferred_element_type=jnp.float32)
        outs[0][...] = e / ssum
        outs[1][...] = tr

    o_spec = pl.BlockSpec((_RB, _W48), lambda i: (i, 0))
    return pl.pallas_call(
        body,
        grid=(_GRID,),
        in_specs=[
            pl.BlockSpec((NC, 3 * _RB, 128), lambda i: (0, i, 0)),
            pl.BlockSpec((_RB, _W48), lambda i: (i, 0)),
            pl.BlockSpec((_RB, 128), lambda i: (i, 0)),
            pl.BlockSpec((1, _W48), lambda i: (0, 0)),
            pl.BlockSpec((_W48, _W48), lambda i: (0, 0)),
            pl.BlockSpec((_W48, _W48), lambda i: (0, 0)),
            pl.BlockSpec((128, _W48), lambda i: (0, 0)),
        ],
        out_specs=[o_spec] * 2,
        out_shape=[jax.ShapeDtypeStruct((_R16, _W48), jnp.float32)] * 2,
    )(p2v, hs2_pk, d16, b2t, seg, s012, m48)


def kernel(x, edge_index, W1, b1, W2, b2):
    ev = edge_index.reshape(2, NCHUNK, CHUNK)
    xv = x.reshape(_XR, 1024)
    eye8 = jnp.eye(8, dtype=jnp.float32)
    bdw1 = jnp.kron(eye8, W1)
    w2p = jnp.zeros((HID, CLSP), jnp.float32).at[:, :CLS].set(W2)
    bdw2 = jnp.kron(eye8, w2p)
    pick = jnp.zeros((HID, CLSP), jnp.float32).at[0, :].set(1.0)
    m48 = jnp.kron(eye8, pick)
    b1t = jnp.tile(b1, 8).reshape(1, 128)
    b2p = jnp.full((CLSP,), -1e30, jnp.float32).at[:CLS].set(b2)
    b2t = jnp.tile(b2p, 8).reshape(1, _W48)
    seg = jnp.kron(eye8, jnp.ones((CLSP, CLSP), jnp.float32))
    ii = jnp.arange(_W48)
    s012 = (ii[None, :] == (3 * (ii % 128) + ii // 128)[:, None]).astype(jnp.float32)

    degp = _degree(ev)
    degv = degp.reshape(NC, _R16, 128)
    hs_pk, d16 = _stage1(xv, bdw1, degv)

    p1 = _propagate16(hs_pk.reshape(NPAD, HID), ev)
    p1v = p1.reshape(NC, _R16, 128)
    hs2_pk = _stage2(p1v, hs_pk, d16, b1t, bdw2, m48)

    p2 = _propagate48(hs2_pk.reshape(NPAD, CLSP), ev)
    p2v = p2.reshape(NC, 3 * _R16, 128)
    out_pk, logit_pk = _stage3(p2v, hs2_pk, d16, b2t, seg, s012, m48)
    out = out_pk.reshape(NPAD, CLSP)[:N, :CLS]
    logits = logit_pk.reshape(NPAD, CLSP)[:N, :CLS]
    return (out, logits)


# speedup vs baseline: 1.2667x; 1.2279x over previous
import jax
import jax.numpy as jnp
from jax import lax
from jax.experimental import pallas as pl
from jax.experimental.pallas import tpu as pltpu
from jax.experimental.pallas import tpu_sc as plsc

N = 10000
E = 320000
F_IN = 128
HID = 16
CLS = 40
CLSP = 48

NC, NS, L = 2, 16, 16
NW = NC * NS
CHUNK = 128
NCHUNK = E // CHUNK
CPWB = NCHUNK // NW
NPAD = 10240
RPT = NPAD // NS


def _sc_mesh():
    return plsc.VectorSubcoreMesh(
        core_axis_name="c", subcore_axis_name="s", num_cores=NC, num_subcores=NS
    )


_SC_PARAMS = pltpu.CompilerParams(use_tc_tiling_on_sc=False)


def _make_propagate(F, nbuf):
    ng = (CPWB + 1 + 2 * nbuf - 1) // (2 * nbuf)

    def body(feat_hbm, edge_hbm, out_hbm, acc, src_v, dst_v, zbuf, rows, gsem, ssem):
        c = lax.axis_index("c")
        s = lax.axis_index("s")
        wid = s * NC + c
        cnt = jnp.where(wid < NCHUNK - NW * CPWB, CPWB + 1, CPWB)
        start = CPWB * wid + jnp.minimum(wid, NCHUNK - NW * CPWB)
        sl = pl.ds(s * RPT, RPT)

        def zfill(i, carry):
            for k in range(F // L):
                zbuf[i, pl.ds(k * L, L)] = jnp.zeros((L,), jnp.float32)
            return carry

        lax.fori_loop(0, 128, zfill, 0)
        for k in range(RPT // 128):
            pltpu.sync_copy(zbuf, acc.at[pl.ds(s * RPT + k * 128, 128)])
        pltpu.sync_copy(edge_hbm.at[0, pl.ds(start, CPWB)], src_v.at[pl.ds(0, CPWB)])
        pltpu.sync_copy(edge_hbm.at[1, pl.ds(start, CPWB)], dst_v.at[pl.ds(0, CPWB)])

        @pl.when(cnt > CPWB)
        def _():
            pltpu.sync_copy(edge_hbm.at[0, start + CPWB], src_v.at[CPWB])
            pltpu.sync_copy(edge_hbm.at[1, start + CPWB], dst_v.at[CPWB])

        plsc.subcore_barrier()

        def super_body(u, carry):
            base = u * 2 * nbuf
            gds = [
                pltpu.async_copy(
                    feat_hbm.at[src_v.at[base + i]], rows[i // nbuf][i % nbuf],
                    gsem[i // nbuf][i % nbuf],
                )
                for i in range(2 * nbuf)
            ]
            sds = []
            for i in range(2 * nbuf):
                gds[i].wait()
                sds.append(
                    pltpu.async_copy(
                        rows[i // nbuf][i % nbuf], acc.at[dst_v.at[base + i]],
                        ssem[i // nbuf][i % nbuf], add=True,
                    )
                )
            for d in sds:
                d.wait()
            return carry

        lax.fori_loop(0, ng - 1, super_body, 0)

        base = (ng - 1) * 2 * nbuf
        gds = [None] * (2 * nbuf)
        for i in range(2 * nbuf):
            def gissue(i=i):
                gds[i] = pltpu.async_copy(
                    feat_hbm.at[src_v.at[base + i]], rows[i // nbuf][i % nbuf],
                    gsem[i // nbuf][i % nbuf],
                )
            pl.when(base + i < cnt)(gissue)
        for i in range(2 * nbuf):
            def gwait(i=i):
                gds[i].wait()
                pltpu.sync_copy(
                    rows[i // nbuf][i % nbuf], acc.at[dst_v.at[base + i]], add=True
                )
            pl.when(base + i < cnt)(gwait)

        plsc.subcore_barrier()
        pltpu.sync_copy(acc.at[sl], out_hbm.at[c, sl])

    return pl.kernel(
        body,
        out_type=jax.ShapeDtypeStruct((NC, NPAD, F), jnp.float32),
        mesh=_sc_mesh(),
        compiler_params=_SC_PARAMS,
        scratch_types=[
            pltpu.VMEM_SHARED((NPAD, F), jnp.float32),
            pltpu.VMEM((ng * 2 * nbuf, CHUNK), jnp.int32),
            pltpu.VMEM((ng * 2 * nbuf, CHUNK), jnp.int32),
            pltpu.VMEM((128, F), jnp.float32),
            [[pltpu.VMEM((CHUNK, F), jnp.float32) for _ in range(nbuf)] for _ in range(2)],
            [[pltpu.SemaphoreType.DMA for _ in range(nbuf)] for _ in range(2)],
            [[pltpu.SemaphoreType.DMA for _ in range(nbuf)] for _ in range(2)],
        ],
    )


def _make_degree():
    def body(edge_hbm, out_hbm, acc, dst_v, ones_v, zbuf):
        c = lax.axis_index("c")
        s = lax.axis_index("s")
        wid = s * NC + c
        sl = pl.ds(s * RPT, RPT)

        def zfill(i, carry):
            zbuf[i, :] = jnp.zeros((L,), jnp.float32)
            return carry

        lax.fori_loop(0, 128, zfill, 0)
        for k in range(RPT // 128):
            pltpu.sync_copy(zbuf, acc.at[pl.ds(s * RPT + k * 128, 128)])

        def ones_body(i, carry):
            ones_v[i, :] = jnp.ones((L,), jnp.float32)
            return carry

        lax.fori_loop(0, CHUNK, ones_body, 0)
        cnt = jnp.where(wid < NCHUNK - NW * CPWB, CPWB + 1, CPWB)
        start = CPWB * wid + jnp.minimum(wid, NCHUNK - NW * CPWB)
        pltpu.sync_copy(edge_hbm.at[1, pl.ds(start, CPWB)], dst_v.at[pl.ds(0, CPWB)])

        @pl.when(cnt > CPWB)
        def _():
            pltpu.sync_copy(edge_hbm.at[1, start + CPWB], dst_v.at[CPWB])

        plsc.subcore_barrier()

        def jbody(j, carry):
            pltpu.sync_copy(ones_v, acc.at[dst_v.at[j]], add=True)
            return carry

        lax.fori_loop(0, CPWB, jbody, 0)

        @pl.when(cnt > CPWB)
        def _():
            pltpu.sync_copy(ones_v, acc.at[dst_v.at[CPWB]], add=True)

        plsc.subcore_barrier()
        pltpu.sync_copy(acc.at[sl], out_hbm.at[c, sl])

    return pl.kernel(
        body,
        out_type=jax.ShapeDtypeStruct((NC, NPAD, L), jnp.float32),
        mesh=_sc_mesh(),
        compiler_params=_SC_PARAMS,
        scratch_types=[
            pltpu.VMEM_SHARED((NPAD, L), jnp.float32),
            pltpu.VMEM((80, CHUNK), jnp.int32),
            pltpu.VMEM((CHUNK, L), jnp.float32),
            pltpu.VMEM((128, L), jnp.float32),
        ],
    )


_propagate16 = _make_propagate(HID, nbuf=4)
_degree = _make_degree()

_RB = 128
_GRID = NPAD // (8 * _RB)
_XR = N * F_IN // 1024
_R16 = NPAD * HID // 128
_W48 = 8 * CLSP


def _stage1(xv, bdw1, degv):
    def body(xv_ref, w_ref, degv_ref, hs_ref, d16_ref):
        d16 = lax.rsqrt(degv_ref[0] + degv_ref[1] + 1.0)
        h = jnp.dot(xv_ref[...], w_ref[...], preferred_element_type=jnp.float32)
        hs_ref[...] = h * d16
        d16_ref[...] = d16

    return pl.pallas_call(
        body,
        grid=(_GRID,),
        in_specs=[
            pl.BlockSpec((_RB, 1024), lambda i: (i, 0)),
            pl.BlockSpec((1024, 128), lambda i: (0, 0)),
            pl.BlockSpec((NC, _RB, 128), lambda i: (0, i, 0)),
        ],
        out_specs=[
            pl.BlockSpec((_RB, 128), lambda i: (i, 0)),
            pl.BlockSpec((_RB, 128), lambda i: (i, 0)),
        ],
        out_shape=[
            jax.ShapeDtypeStruct((_R16, 128), jnp.float32),
            jax.ShapeDtypeStruct((_R16, 128), jnp.float32),
        ],
    )(xv, bdw1, degv)


def _stage2(p1v, hs_pk, d16, b1t):
    def body(p_ref, hs_ref, d16_ref, b1_ref, h1s_ref):
        d16 = d16_ref[...]
        sacc = d16 * (p_ref[0] + p_ref[1] + hs_ref[...]) + b1_ref[...]
        h1s_ref[...] = jnp.maximum(sacc, 0.0) * d16

    return pl.pallas_call(
        body,
        grid=(_GRID,),
        in_specs=[
            pl.BlockSpec((NC, _RB, 128), lambda i: (0, i, 0)),
            pl.BlockSpec((_RB, 128), lambda i: (i, 0)),
            pl.BlockSpec((_RB, 128), lambda i: (i, 0)),
            pl.BlockSpec((1, 128), lambda i: (0, 0)),
        ],
        out_specs=pl.BlockSpec((_RB, 128), lambda i: (i, 0)),
        out_shape=jax.ShapeDtypeStruct((_R16, 128), jnp.float32),
    )(p1v, hs_pk, d16, b1t)


def _stage3(p2v, h1s, d16, b2t, seg, bdw2):
    def body(p_ref, h1s_ref, d16_ref, b2_ref, seg_ref, w_ref, *outs):
        q = d16_ref[...] * (p_ref[0] + p_ref[1] + h1s_ref[...])
        tr = jnp.dot(q, w_ref[...], preferred_element_type=jnp.float32) + b2_ref[...]
        m = jnp.max(tr, axis=1, keepdims=True)
        e = jnp.exp(tr - m)
        ssum = jnp.dot(e, seg_ref[...], preferred_element_type=jnp.float32)
        outs[0][...] = e / ssum
        outs[1][...] = tr

    o_spec = pl.BlockSpec((_RB, _W48), lambda i: (i, 0))
    return pl.pallas_call(
        body,
        grid=(_GRID,),
        in_specs=[
            pl.BlockSpec((NC, _RB, 128), lambda i: (0, i, 0)),
            pl.BlockSpec((_RB, 128), lambda i: (i, 0)),
            pl.BlockSpec((_RB, 128), lambda i: (i, 0)),
            pl.BlockSpec((1, _W48), lambda i: (0, 0)),
            pl.BlockSpec((_W48, _W48), lambda i: (0, 0)),
            pl.BlockSpec((128, _W48), lambda i: (0, 0)),
        ],
        out_specs=[o_spec] * 2,
        out_shape=[jax.ShapeDtypeStruct((_R16, _W48), jnp.float32)] * 2,
    )(p2v, h1s, d16, b2t, seg, bdw2)


def kernel(x, edge_index, W1, b1, W2, b2):
    ev = edge_index.reshape(2, NCHUNK, CHUNK)
    xv = x.reshape(_XR, 1024)
    eye8 = jnp.eye(8, dtype=jnp.float32)
    bdw1 = jnp.kron(eye8, W1)
    w2p = jnp.zeros((HID, CLSP), jnp.float32).at[:, :CLS].set(W2)
    bdw2 = jnp.kron(eye8, w2p)
    b1t = jnp.tile(b1, 8).reshape(1, 128)
    b2p = jnp.full((CLSP,), -1e30, jnp.float32).at[:CLS].set(b2)
    b2t = jnp.tile(b2p, 8).reshape(1, _W48)
    seg = jnp.kron(eye8, jnp.ones((CLSP, CLSP), jnp.float32))

    degp = _degree(ev)
    degv = degp.reshape(NC, _R16, 128)
    hs_pk, d16 = _stage1(xv, bdw1, degv)

    p1 = _propagate16(hs_pk.reshape(NPAD, HID), ev)
    p1v = p1.reshape(NC, _R16, 128)
    h1s = _stage2(p1v, hs_pk, d16, b1t)

    p2 = _propagate16(h1s.reshape(NPAD, HID), ev)
    p2v = p2.reshape(NC, _R16, 128)
    out_pk, logit_pk = _stage3(p2v, h1s, d16, b2t, seg, bdw2)
    out = out_pk.reshape(NPAD, CLSP)[:N, :CLS]
    logits = logit_pk.reshape(NPAD, CLSP)[:N, :CLS]
    return (out, logits)
